# baseline (speedup 1.0000x reference)
import functools

import jax
import jax.numpy as jnp
from jax import lax
from jax.experimental import pallas as pl
from jax.experimental.pallas import tpu as pltpu
from jax.experimental.pallas import tpu_sc as plsc

B = 16384
D = 16
NT = 14
F1 = 128

NC = 2
NS = 16
NW = NC * NS
BPW = B // NW


_sc_mesh = plsc.VectorSubcoreMesh(core_axis_name="c", subcore_axis_name="s")


@functools.partial(
    pl.kernel,
    out_type=jax.ShapeDtypeStruct((B, NT, D), jnp.float32),
    mesh=_sc_mesh,
    scratch_types=[
        pltpu.VMEM((NT, BPW), jnp.int32),
        pltpu.VMEM((NT, BPW, D), jnp.float32),
        pltpu.SemaphoreType.DMA,
    ],
    compiler_params=pltpu.CompilerParams(use_tc_tiling_on_sc=False),
)
def _sc_gather(*refs):
    tables = refs[:NT]
    idxs = refs[NT:2 * NT]
    out_hbm = refs[2 * NT]
    idx_v, rows_v, sem = refs[2 * NT + 1:]

    wid = lax.axis_index("s") * NC + lax.axis_index("c")
    base = wid * BPW

    idx_copies = [
        pltpu.async_copy(idxs[t].at[pl.ds(base, BPW)], idx_v.at[t], sem)
        for t in range(NT)
    ]
    for c in idx_copies:
        c.wait()

    gathers = [
        pltpu.async_copy(tables[t].at[idx_v.at[t]], rows_v.at[t], sem)
        for t in range(NT)
    ]
    for c in gathers:
        c.wait()

    out_copies = [
        pltpu.async_copy(rows_v.at[t], out_hbm.at[pl.ds(base, BPW), t], sem)
        for t in range(NT)
    ]
    for c in out_copies:
        c.wait()


BLK = 2048


def _mlp_body(emb_ref, price_ref, w1t_ref, w1p_ref, b1_ref, w2t_ref, b2_ref,
              out_ref):
    x = emb_ref[...]
    fc1 = lax.dot_general(
        x, w1t_ref[...], (((1,), (0,)), ((), ())),
        preferred_element_type=jnp.float32,
        precision=lax.Precision.HIGHEST)
    fc1 = fc1 + price_ref[...] * w1p_ref[...] + b1_ref[...]
    fc1 = jnp.maximum(fc1, 0.0)
    z = lax.dot_general(
        fc1, w2t_ref[...], (((1,), (0,)), ((), ())),
        preferred_element_type=jnp.float32,
        precision=lax.Precision.HIGHEST)
    z = z + b2_ref[...]
    out_ref[...] = 1.0 / (1.0 + jnp.exp(-z))


_mlp = pl.pallas_call(
    _mlp_body,
    grid=(B // BLK,),
    in_specs=[
        pl.BlockSpec((BLK, NT * D), lambda i: (i, 0)),
        pl.BlockSpec((BLK, 1), lambda i: (i, 0)),
        pl.BlockSpec((NT * D, F1), lambda i: (0, 0)),
        pl.BlockSpec((1, F1), lambda i: (0, 0)),
        pl.BlockSpec((1, F1), lambda i: (0, 0)),
        pl.BlockSpec((F1, 1), lambda i: (0, 0)),
        pl.BlockSpec((1, 1), lambda i: (0, 0)),
    ],
    out_specs=pl.BlockSpec((BLK, 1), lambda i: (i, 0)),
    out_shape=jax.ShapeDtypeStruct((B, 1), jnp.float32),
)


def kernel(userId, cmsSegId, cmsGroupId, finalGenderCode, ageLevel,
           pvalueLevel, shoppingLevel, occupation, newUserClassLevel,
           adGroupId, cateId, campaignId, customer, brand, price,
           userId_table, cmsSegId_table, cmsGroupId_table,
           finalGenderCode_table, ageLevel_table, pvalueLevel_table,
           shoppingLevel_table, occupation_table, newUserClassLevel_table,
           adGroupId_table, cateId_table, campaignId_table, customer_table,
           brand_table, W1, b1, W2, b2):
    tables = (userId_table, adGroupId_table, cmsSegId_table, cmsGroupId_table,
              finalGenderCode_table, ageLevel_table, pvalueLevel_table,
              shoppingLevel_table, occupation_table, newUserClassLevel_table,
              cateId_table, campaignId_table, customer_table, brand_table)
    idxs = (userId, adGroupId, cmsSegId, cmsGroupId, finalGenderCode,
            ageLevel, pvalueLevel, shoppingLevel, occupation,
            newUserClassLevel, cateId, campaignId, customer, brand)
    idxs = tuple(i.reshape(B) for i in idxs)

    emb = _sc_gather(*tables, *idxs)
    emb = emb.reshape(B, NT * D)

    w1t = W1[:, :NT * D].T
    w1p = W1[:, NT * D].reshape(1, F1)
    return _mlp(emb, price, w1t, w1p, b1.reshape(1, F1), W2.T,
                b2.reshape(1, 1))
